# Initial kernel scaffold; baseline (speedup 1.0000x reference)
import jax
import jax.numpy as jnp
from jax.experimental import pallas as pl


def kernel(inputs, call, stand, taxi, hr, week, callemb, standemb, taxiemb, hremb, weekemb, W, b):
    raise NotImplementedError("write your pallas kernel here")


# speedup vs baseline: 2.6856x; 2.6856x over previous
import functools

import jax
import jax.numpy as jnp
from jax import lax
from jax.experimental import pallas as pl
from jax.experimental.pallas import tpu as pltpu
from jax.experimental.pallas import tpu_sc as plsc

B = 16384
ENUM = 32
IN_FEATS = 80

STAND_OFF = 0
TAXI_OFF = 72
HR_OFF = 1080
WEEK_OFF = 1128
PS_ROWS = 1136

NC, NS, LANES = 2, 16, 16
NW = NC * NS
ROWS_PER_W = B // NW
GROUPS = ROWS_PER_W // LANES
IDX_CHUNK = 128
N_CHUNKS = ROWS_PER_W // IDX_CHUNK


def _tc_body(x_ref, wi_ref, b_ref, st_ref, ws_ref, tx_ref, wt_ref,
             hr_ref, wh_ref, wk_ref, ww_ref, dense_ref, ps_ref):
    dense_ref[...] = (
        jnp.dot(x_ref[...], wi_ref[...], preferred_element_type=jnp.float32)
        + b_ref[...]
    )
    ps_ref[0:72, :] = jnp.dot(st_ref[...], ws_ref[...],
                              preferred_element_type=jnp.float32)
    ps_ref[72:1080, :] = jnp.dot(tx_ref[...], wt_ref[...],
                                 preferred_element_type=jnp.float32)
    ps_ref[1080:1128, :] = jnp.dot(hr_ref[...], wh_ref[...],
                                   preferred_element_type=jnp.float32)
    ps_ref[1128:1136, :] = jnp.dot(wk_ref[...], ww_ref[...],
                                   preferred_element_type=jnp.float32)


def _sc_body(dense_hbm, callidx_hbm, stand_hbm, taxi_hbm, hr_hbm, week_hbm,
             ps_hbm, wcb_hbm, callemb_hbm, out_hbm,
             call_v, stand_v, taxi_v, hr_v, week_v,
             dense_v, ps_v, wcb_v, rows_v, out_v, sem):
    wid = lax.axis_index("s") * NC + lax.axis_index("c")
    base = wid * ROWS_PER_W

    pltpu.sync_copy(callidx_hbm.at[wid], call_v)
    pltpu.sync_copy(stand_hbm.at[pl.ds(base, ROWS_PER_W)], stand_v)
    pltpu.sync_copy(taxi_hbm.at[pl.ds(base, ROWS_PER_W)], taxi_v)
    pltpu.sync_copy(hr_hbm.at[pl.ds(base, ROWS_PER_W)], hr_v)
    pltpu.sync_copy(week_hbm.at[pl.ds(base, ROWS_PER_W)], week_v)
    pltpu.sync_copy(dense_hbm.at[pl.ds(base * 2, 2 * ROWS_PER_W)], dense_v)
    pltpu.sync_copy(ps_hbm, ps_v)
    pltpu.sync_copy(wcb_hbm, wcb_v)

    copies = [
        pltpu.async_copy(
            callemb_hbm.at[call_v.at[j]],
            rows_v.at[j],
            sem,
        )
        for j in range(N_CHUNKS)
    ]
    for c in copies:
        c.wait()

    lanes = lax.iota(jnp.int32, LANES)
    lanes2 = lanes * 2

    def group(g, carry):
        out_base = lanes2 + g * (2 * LANES)
        acc0 = plsc.load_gather(dense_v, [out_base])
        acc1 = plsc.load_gather(dense_v, [out_base + 1])
        for idx_ref in (stand_v, taxi_v, hr_v, week_v):
            iv = idx_ref[pl.ds(g * LANES, LANES)]
            acc0 = acc0 + plsc.load_gather(ps_v, [iv])
            acc1 = acc1 + plsc.load_gather(ps_v, [iv + 1])
        jv = jnp.full((LANES,), g // 8, jnp.int32)
        rv = lanes + (g % 8) * LANES
        for f in range(ENUM):
            x = plsc.load_gather(rows_v, [jv, rv, jnp.full((LANES,), f, jnp.int32)])
            w0 = wcb_v[pl.ds(f * LANES, LANES)]
            w1 = wcb_v[pl.ds(ROWS_PER_W + f * LANES, LANES)]
            acc0 = acc0 + x * w0
            acc1 = acc1 + x * w1
        plsc.store_scatter(out_v, [out_base], acc0)
        plsc.store_scatter(out_v, [out_base + 1], acc1)
        return carry

    lax.fori_loop(0, GROUPS, group, 0)
    pltpu.sync_copy(out_v, out_hbm.at[pl.ds(base * 2, 2 * ROWS_PER_W)])


_sc_call = functools.partial(
    pl.kernel,
    out_type=jax.ShapeDtypeStruct((B * 2,), jnp.float32),
    compiler_params=pltpu.CompilerParams(
        needs_layout_passes=False, use_tc_tiling_on_sc=False,
    ),
    mesh=plsc.VectorSubcoreMesh(
        core_axis_name="c", subcore_axis_name="s",
        num_cores=NC, num_subcores=NS,
    ),
    scratch_types=[
        pltpu.VMEM((N_CHUNKS, IDX_CHUNK), jnp.int32),
        pltpu.VMEM((ROWS_PER_W,), jnp.int32),
        pltpu.VMEM((ROWS_PER_W,), jnp.int32),
        pltpu.VMEM((ROWS_PER_W,), jnp.int32),
        pltpu.VMEM((ROWS_PER_W,), jnp.int32),
        pltpu.VMEM((2 * ROWS_PER_W,), jnp.float32),
        pltpu.VMEM((2 * PS_ROWS,), jnp.float32),
        pltpu.VMEM((2 * ROWS_PER_W,), jnp.float32),
        pltpu.VMEM((N_CHUNKS, IDX_CHUNK, ENUM), jnp.float32),
        pltpu.VMEM((2 * ROWS_PER_W,), jnp.float32),
        pltpu.SemaphoreType.DMA,
    ],
)(_sc_body)


def kernel(inputs, call, stand, taxi, hr, week, callemb, standemb, taxiemb,
           hremb, weekemb, W, b):
    f32 = jnp.float32
    i32 = jnp.int32
    wi = W[:, :IN_FEATS].T
    wc = W[:, IN_FEATS:IN_FEATS + ENUM]
    ws = W[:, 112:144].T
    wt = W[:, 144:176].T
    wh = W[:, 176:208].T
    ww = W[:, 208:240].T
    wcb = jnp.repeat(wc, LANES, axis=1).reshape(-1)

    st_p = jnp.pad(standemb, ((0, 72 - 65), (0, 0)))
    tx_p = jnp.pad(taxiemb, ((0, 1008 - 1000), (0, 0)))
    wk_p = jnp.pad(weekemb, ((0, 8 - 7), (0, 0)))

    dense, ps = pl.pallas_call(
        _tc_body,
        out_shape=[
            jax.ShapeDtypeStruct((B, 2), f32),
            jax.ShapeDtypeStruct((PS_ROWS, 2), f32),
        ],
    )(inputs, wi, b.reshape(1, 2), st_p, ws, tx_p, wt, hremb, wh, wk_p, ww)

    callidx = (call.astype(i32) + 1).reshape(NW, N_CHUNKS, IDX_CHUNK)
    standg = (stand.astype(i32) + (1 + STAND_OFF)) * 2
    taxig = (taxi.astype(i32) + TAXI_OFF) * 2
    hrg = (hr.astype(i32) + HR_OFF) * 2
    weekg = (week.astype(i32) + WEEK_OFF) * 2

    out_flat = _sc_call(dense.reshape(-1), callidx, standg, taxig, hrg, weekg,
                        ps.reshape(-1), wcb, callemb)
    return out_flat.reshape(B, 2)
